# baseline (speedup 1.0000x reference)
import functools

import jax
import jax.numpy as jnp
from jax import lax
from jax.experimental import pallas as pl
from jax.experimental.pallas import tpu as pltpu
from jax.experimental.pallas import tpu_sc as plsc

_B, _C, _D, _H, _W = 2, 4, 128, 128, 128
_BC = _B * _C
_ROWS = _D * _H
_SPATIAL = _D * _H * _W
_N_TOP = int(round(_SPATIAL * 10 / 100))

_RBLK = 2048

_NW = 32
_WPR = 4
_EPW = _SPATIAL // _WPR
_CHUNK = 16384
_NCHUNK = _EPW // _CHUNK

_NBINS1 = 256
_NBINS2 = 2048


_ZBLK = 16


def _loss_body(bbox_ref, net_ref, tgt_ref, out_ref, e8_ref):
    bc = pl.program_id(0)
    i = pl.program_id(1)
    x = net_ref[0]
    t = tgt_ref[0]
    zc = i * _ZBLK + lax.broadcasted_iota(jnp.int32, (_ZBLK, 1, 1), 0)
    yc = lax.broadcasted_iota(jnp.int32, (1, _H, 1), 1)
    xc = lax.broadcasted_iota(jnp.int32, (1, 1, _W), 2)
    zm = (zc >= bbox_ref[bc, 0]) & (zc < bbox_ref[bc, 1])
    ym = (yc >= bbox_ref[bc, 2]) & (yc < bbox_ref[bc, 3])
    xm = (xc >= bbox_ref[bc, 4]) & (xc < bbox_ref[bc, 5])
    dummy = jnp.where((zm & ym) & xm, t, jnp.float32(0.0))
    loss = (jnp.maximum(x, 0.0) - x * dummy
            + jnp.log1p(jnp.exp(-jnp.abs(x))))
    out_ref[0] = loss
    key = lax.bitcast_convert_type(loss, jnp.int32)
    e8_ref[0] = lax.shift_right_logical(key, 23).astype(jnp.uint8)


def _compute_loss(net4, tgt4, bbox6):
    grid_spec = pltpu.PrefetchScalarGridSpec(
        num_scalar_prefetch=1,
        grid=(_BC, _D // _ZBLK),
        in_specs=[
            pl.BlockSpec((1, _ZBLK, _H, _W), lambda bc, i, bb: (bc, i, 0, 0)),
            pl.BlockSpec((1, _ZBLK, _H, _W),
                         lambda bc, i, bb: (bc // _C, i, 0, 0)),
        ],
        out_specs=[
            pl.BlockSpec((1, _ZBLK, _H, _W), lambda bc, i, bb: (bc, i, 0, 0)),
            pl.BlockSpec((1, _ZBLK, _H, _W), lambda bc, i, bb: (bc, i, 0, 0)),
        ],
    )
    return pl.pallas_call(
        _loss_body,
        grid_spec=grid_spec,
        out_shape=[
            jax.ShapeDtypeStruct((_BC, _D, _H, _W), jnp.float32),
            jax.ShapeDtypeStruct((_BC, _D, _H, _W), jnp.uint8),
        ],
    )(bbox6, net4, tgt4)


def _make_sc_hist(level2):
    nbins = 2049 if level2 else 256
    outrow = 2056 if level2 else 256
    buf_dtype = jnp.float32 if level2 else jnp.int32
    epw = _EPW if level2 else _EPW // 4
    spat = _SPATIAL if level2 else _SPATIAL // 4
    nchunk = epw // _CHUNK
    stride = 2051 if level2 else 257
    nb16 = stride * 16 + 16
    ngrp = (nbins + 15) // 16
    mesh = plsc.VectorSubcoreMesh(core_axis_name="c", subcore_axis_name="s")

    scratch = [
        pltpu.VMEM((_CHUNK,), buf_dtype),
        pltpu.VMEM((_CHUNK,), buf_dtype),
        pltpu.VMEM((nb16,), jnp.float32),
        pltpu.VMEM((nb16,), jnp.float32),
        pltpu.VMEM((16,), jnp.int32),
        pltpu.VMEM((16,), jnp.int32),
        pltpu.SemaphoreType.DMA,
        pltpu.SemaphoreType.DMA,
    ]
    if level2:
        out_type = (
            jax.ShapeDtypeStruct((_NW * outrow,), jnp.float32),
            jax.ShapeDtypeStruct((_NW * outrow,), jnp.float32),
        )
    else:
        out_type = jax.ShapeDtypeStruct((_NW * outrow,), jnp.float32)

    def body(loss_hbm, klo_hbm, khi_hbm, *rest):
        if level2:
            cnt_hbm, sum_hbm = rest[:2]
            rest = rest[2:]
        else:
            cnt_hbm = rest[0]
            rest = rest[1:]
        buf_a, buf_b, hc, hs, klo_v, khi_v, sem_a, sem_b = rest
        bufs = (buf_a, buf_b)
        sems = (sem_a, sem_b)
        w = lax.axis_index("s") * 2 + lax.axis_index("c")
        row = w // _WPR
        base = row * spat + (w % _WPR) * epw
        lane = lax.iota(jnp.int32, 16)
        lane_off = lane * stride
        zeros = jnp.zeros((16,), jnp.float32)
        ones = jnp.ones((16,), jnp.float32)

        def zinit(j, _):
            hc[pl.ds(j * 16, 16)] = zeros
            hs[pl.ds(j * 16, 16)] = zeros
            return 0
        lax.fori_loop(0, nb16 // 16, zinit, 0)

        if level2:
            pltpu.sync_copy(klo_hbm.at[pl.ds(row * 16, 16)], klo_v)
            pltpu.sync_copy(khi_hbm.at[pl.ds(row * 16, 16)], khi_v)
            klo = klo_v[...]
            khi = khi_v[...]
            ovf = jnp.full((16,), 2048, jnp.int32)

        def copy(ci, b):
            return pltpu.make_async_copy(
                loss_hbm.at[pl.ds(base + ci * _CHUNK, _CHUNK)],
                bufs[b], sems[b])

        copy(0, 0).start()
        copy(1, 1).start()

        def outer(g, _):
            for b in range(2):
                ci = g * 2 + b
                copy(ci, b).wait()
                buf = bufs[b]

                @plsc.parallel_loop(0, _CHUNK // 16, unroll=8)
                def vreg_body(vi):
                    v = buf[pl.ds(vi * 16, 16)]
                    if level2:
                        k = lax.bitcast_convert_type(v, jnp.int32)
                        sub = lax.shift_right_logical(k, 12) & 2047
                        idx = jnp.where(k < khi, sub, ovf)
                        fidx = idx + lane_off
                        msk = k >= klo
                        plsc.addupdate_scatter(hc, [fidx], ones, mask=msk)
                        plsc.addupdate_scatter(hs, [fidx], v, mask=msk)
                    else:
                        for sh in (0, 8, 16, 24):
                            idx = lax.shift_right_logical(v, sh) & 255
                            plsc.addupdate_scatter(hc, [idx + lane_off],
                                                   ones)

                @pl.when(ci + 2 < nchunk)
                def _():
                    copy(ci + 2, b).start()
            return 0
        lax.fori_loop(0, nchunk // 2, outer, 0)

        def comb(j, _):
            def inner(l, accs):
                off = l * stride + j * 16
                return (accs[0] + hc[pl.ds(off, 16)],
                        accs[1] + hs[pl.ds(off, 16)])
            acc_c, acc_s = lax.fori_loop(0, 16, inner, (zeros, zeros))
            hc[pl.ds(j * 16, 16)] = acc_c
            if level2:
                hs[pl.ds(j * 16, 16)] = acc_s
            return 0
        lax.fori_loop(0, ngrp, comb, 0)

        pltpu.sync_copy(hc.at[pl.ds(0, outrow)],
                        cnt_hbm.at[pl.ds(w * outrow, outrow)])
        if level2:
            pltpu.sync_copy(hs.at[pl.ds(0, outrow)],
                            sum_hbm.at[pl.ds(w * outrow, outrow)])

    return pl.kernel(
        body, out_type=out_type, mesh=mesh, scratch_types=scratch,
        compiler_params=pltpu.CompilerParams(needs_layout_passes=False))


def _rev_cumsum(a):
    return jnp.cumsum(a[:, ::-1], axis=1)[:, ::-1]


def kernel(net_output, target_structure, bboxes):
    net4 = net_output.reshape(_BC, _D, _H, _W)
    tgt4 = target_structure.reshape(_B, _D, _H, _W)
    bbox6 = bboxes.astype(jnp.int32).reshape(_BC, 6)

    loss, e8 = _compute_loss(net4, tgt4, bbox6)
    loss_flat = loss.reshape(-1)
    key_words = lax.bitcast_convert_type(e8.reshape(-1, 4), jnp.int32)

    dummy_bounds = jnp.zeros((_BC * 16,), jnp.int32)

    cnt1 = _make_sc_hist(False)(key_words, dummy_bounds, dummy_bounds)
    cnt1 = cnt1.reshape(_BC, _WPR, _NBINS1).sum(1).astype(jnp.int32)

    n = jnp.int32(_N_TOP)
    cum1 = _rev_cumsum(cnt1)
    b1 = jnp.sum(cum1 >= n, axis=1) - 1
    klo = b1 << 23
    khi = (b1 + 1) << 23

    klo_v = jnp.broadcast_to(klo[:, None], (_BC, 16)).reshape(-1)
    khi_v = jnp.broadcast_to(khi[:, None], (_BC, 16)).reshape(-1)

    cnt2p, sum2p = _make_sc_hist(True)(loss_flat, klo_v, khi_v)
    cnt2p = cnt2p.reshape(_BC, _WPR, 2056).sum(1)
    sum2p = sum2p.reshape(_BC, _WPR, 2056).sum(1)
    cnt2 = cnt2p[:, :_NBINS2].astype(jnp.int32)
    sum2 = sum2p[:, :_NBINS2]
    c1_above = cnt2p[:, _NBINS2].astype(jnp.int32)
    s1_above = sum2p[:, _NBINS2]

    n2 = n - c1_above
    cum2 = _rev_cumsum(cnt2)
    b2 = jnp.sum(cum2 >= n2[:, None], axis=1) - 1
    take2 = lambda a: jnp.take_along_axis(a, b2[:, None], axis=1)[:, 0]
    c2_above = take2(cum2) - take2(cnt2)
    s2_above = take2(_rev_cumsum(sum2)) - take2(sum2)
    ties = (n2 - c2_above).astype(jnp.float32)
    tie_mean = take2(sum2) / jnp.maximum(take2(cnt2), 1).astype(jnp.float32)

    row_total = s1_above + s2_above + ties * tie_mean
    return (jnp.sum(row_total) / jnp.float32(_BC * _N_TOP)).astype(jnp.float32)


# speedup vs baseline: 12.3078x; 12.3078x over previous
import functools

import jax
import jax.numpy as jnp
from jax import lax
from jax.experimental import pallas as pl
from jax.experimental.pallas import tpu as pltpu
from jax.experimental.pallas import tpu_sc as plsc

_B, _C, _D, _H, _W = 2, 4, 128, 128, 128
_BC = _B * _C
_ROWS = _D * _H
_SPATIAL = _D * _H * _W
_N_TOP = int(round(_SPATIAL * 10 / 100))

_RBLK = 2048

_NW = 32
_WPR = 4
_EPW = _SPATIAL // _WPR
_CHUNK = 16384
_NCHUNK = _EPW // _CHUNK

_NBINS1 = 256
_NBINS2 = 2048


_ZBLK = 16


def _loss_body(bbox_ref, net_ref, tgt_ref, out_ref, e8_ref):
    bc = pl.program_id(0)
    i = pl.program_id(1)
    x = net_ref[0]
    t = tgt_ref[0]
    zc = i * _ZBLK + lax.broadcasted_iota(jnp.int32, (_ZBLK, 1, 1), 0)
    yc = lax.broadcasted_iota(jnp.int32, (1, _H, 1), 1)
    xc = lax.broadcasted_iota(jnp.int32, (1, 1, _W), 2)
    zm = (zc >= bbox_ref[bc, 0]) & (zc < bbox_ref[bc, 1])
    ym = (yc >= bbox_ref[bc, 2]) & (yc < bbox_ref[bc, 3])
    xm = (xc >= bbox_ref[bc, 4]) & (xc < bbox_ref[bc, 5])
    dummy = jnp.where((zm & ym) & xm, t, jnp.float32(0.0))
    loss = (jnp.maximum(x, 0.0) - x * dummy
            + jnp.log1p(jnp.exp(-jnp.abs(x))))
    out_ref[0] = loss
    key = lax.bitcast_convert_type(loss, jnp.int32)
    e = lax.shift_right_logical(key, 23).reshape(_ZBLK // 4, 4, _H, _W)
    e8_ref[0] = (e[:, 0] | (e[:, 1] << 8) | (e[:, 2] << 16)
                 | (e[:, 3] << 24))


def _compute_loss(net4, tgt4, bbox6):
    grid_spec = pltpu.PrefetchScalarGridSpec(
        num_scalar_prefetch=1,
        grid=(_BC, _D // _ZBLK),
        in_specs=[
            pl.BlockSpec((1, _ZBLK, _H, _W), lambda bc, i, bb: (bc, i, 0, 0)),
            pl.BlockSpec((1, _ZBLK, _H, _W),
                         lambda bc, i, bb: (bc // _C, i, 0, 0)),
        ],
        out_specs=[
            pl.BlockSpec((1, _ZBLK, _H, _W), lambda bc, i, bb: (bc, i, 0, 0)),
            pl.BlockSpec((1, _ZBLK // 4, _H, _W),
                         lambda bc, i, bb: (bc, i, 0, 0)),
        ],
    )
    return pl.pallas_call(
        _loss_body,
        grid_spec=grid_spec,
        out_shape=[
            jax.ShapeDtypeStruct((_BC, _D, _H, _W), jnp.float32),
            jax.ShapeDtypeStruct((_BC, _D // 4, _H, _W), jnp.int32),
        ],
    )(bbox6, net4, tgt4)


def _make_sc_hist(level2):
    nbins = 2049 if level2 else 256
    outrow = 2056 if level2 else 256
    buf_dtype = jnp.float32 if level2 else jnp.int32
    epw = _EPW if level2 else _EPW // 4
    spat = _SPATIAL if level2 else _SPATIAL // 4
    nchunk = epw // _CHUNK
    stride = 2051 if level2 else 257
    nb16 = stride * 16 + 16
    ngrp = (nbins + 15) // 16
    mesh = plsc.VectorSubcoreMesh(core_axis_name="c", subcore_axis_name="s")

    scratch = [
        pltpu.VMEM((_CHUNK,), buf_dtype),
        pltpu.VMEM((_CHUNK,), buf_dtype),
        pltpu.VMEM((nb16,), jnp.float32),
        pltpu.VMEM((nb16,), jnp.float32),
        pltpu.VMEM((16,), jnp.int32),
        pltpu.VMEM((16,), jnp.int32),
        pltpu.SemaphoreType.DMA,
        pltpu.SemaphoreType.DMA,
    ]
    if level2:
        out_type = (
            jax.ShapeDtypeStruct((_NW * outrow,), jnp.float32),
            jax.ShapeDtypeStruct((_NW * outrow,), jnp.float32),
        )
    else:
        out_type = jax.ShapeDtypeStruct((_NW * outrow,), jnp.float32)

    def body(loss_hbm, klo_hbm, khi_hbm, *rest):
        if level2:
            cnt_hbm, sum_hbm = rest[:2]
            rest = rest[2:]
        else:
            cnt_hbm = rest[0]
            rest = rest[1:]
        buf_a, buf_b, hc, hs, klo_v, khi_v, sem_a, sem_b = rest
        bufs = (buf_a, buf_b)
        sems = (sem_a, sem_b)
        w = lax.axis_index("s") * 2 + lax.axis_index("c")
        row = w // _WPR
        base = row * spat + (w % _WPR) * epw
        lane = lax.iota(jnp.int32, 16)
        lane_off = lane * stride
        zeros = jnp.zeros((16,), jnp.float32)
        ones = jnp.ones((16,), jnp.float32)

        def zinit(j, _):
            hc[pl.ds(j * 16, 16)] = zeros
            hs[pl.ds(j * 16, 16)] = zeros
            return 0
        lax.fori_loop(0, nb16 // 16, zinit, 0)

        if level2:
            pltpu.sync_copy(klo_hbm.at[pl.ds(row * 16, 16)], klo_v)
            pltpu.sync_copy(khi_hbm.at[pl.ds(row * 16, 16)], khi_v)
            klo = klo_v[...]
            khi = khi_v[...]
            ovf = jnp.full((16,), 2048, jnp.int32)

        def copy(ci, b):
            return pltpu.make_async_copy(
                loss_hbm.at[pl.ds(base + ci * _CHUNK, _CHUNK)],
                bufs[b], sems[b])

        copy(0, 0).start()
        copy(1, 1).start()

        def outer(g, _):
            for b in range(2):
                ci = g * 2 + b
                copy(ci, b).wait()
                buf = bufs[b]

                @plsc.parallel_loop(0, _CHUNK // 16, unroll=8)
                def vreg_body(vi):
                    v = buf[pl.ds(vi * 16, 16)]
                    if level2:
                        k = lax.bitcast_convert_type(v, jnp.int32)
                        sub = lax.shift_right_logical(k, 12) & 2047
                        idx = jnp.where(k < khi, sub, ovf)
                        fidx = idx + lane_off
                        msk = k >= klo
                        plsc.addupdate_scatter(hc, [fidx], ones, mask=msk)
                        plsc.addupdate_scatter(hs, [fidx], v, mask=msk)
                    else:
                        for sh in (0, 8, 16, 24):
                            idx = lax.shift_right_logical(v, sh) & 255
                            plsc.addupdate_scatter(hc, [idx + lane_off],
                                                   ones)

                @pl.when(ci + 2 < nchunk)
                def _():
                    copy(ci + 2, b).start()
            return 0
        lax.fori_loop(0, nchunk // 2, outer, 0)

        def comb(j, _):
            def inner(l, accs):
                off = l * stride + j * 16
                return (accs[0] + hc[pl.ds(off, 16)],
                        accs[1] + hs[pl.ds(off, 16)])
            acc_c, acc_s = lax.fori_loop(0, 16, inner, (zeros, zeros))
            hc[pl.ds(j * 16, 16)] = acc_c
            if level2:
                hs[pl.ds(j * 16, 16)] = acc_s
            return 0
        lax.fori_loop(0, ngrp, comb, 0)

        pltpu.sync_copy(hc.at[pl.ds(0, outrow)],
                        cnt_hbm.at[pl.ds(w * outrow, outrow)])
        if level2:
            pltpu.sync_copy(hs.at[pl.ds(0, outrow)],
                            sum_hbm.at[pl.ds(w * outrow, outrow)])

    return pl.kernel(
        body, out_type=out_type, mesh=mesh, scratch_types=scratch,
        compiler_params=pltpu.CompilerParams(needs_layout_passes=False))


def _rev_cumsum(a):
    return jnp.cumsum(a[:, ::-1], axis=1)[:, ::-1]


def kernel(net_output, target_structure, bboxes):
    net4 = net_output.reshape(_BC, _D, _H, _W)
    tgt4 = target_structure.reshape(_B, _D, _H, _W)
    bbox6 = bboxes.astype(jnp.int32).reshape(_BC, 6)

    loss, ew = _compute_loss(net4, tgt4, bbox6)
    loss_flat = loss.reshape(-1)
    key_words = ew.reshape(-1)

    dummy_bounds = jnp.zeros((_BC * 16,), jnp.int32)

    cnt1 = _make_sc_hist(False)(key_words, dummy_bounds, dummy_bounds)
    cnt1 = cnt1.reshape(_BC, _WPR, _NBINS1).sum(1).astype(jnp.int32)

    n = jnp.int32(_N_TOP)
    cum1 = _rev_cumsum(cnt1)
    b1 = jnp.sum(cum1 >= n, axis=1) - 1
    klo = b1 << 23
    khi = (b1 + 1) << 23

    klo_v = jnp.broadcast_to(klo[:, None], (_BC, 16)).reshape(-1)
    khi_v = jnp.broadcast_to(khi[:, None], (_BC, 16)).reshape(-1)

    cnt2p, sum2p = _make_sc_hist(True)(loss_flat, klo_v, khi_v)
    cnt2p = cnt2p.reshape(_BC, _WPR, 2056).sum(1)
    sum2p = sum2p.reshape(_BC, _WPR, 2056).sum(1)
    cnt2 = cnt2p[:, :_NBINS2].astype(jnp.int32)
    sum2 = sum2p[:, :_NBINS2]
    c1_above = cnt2p[:, _NBINS2].astype(jnp.int32)
    s1_above = sum2p[:, _NBINS2]

    n2 = n - c1_above
    cum2 = _rev_cumsum(cnt2)
    b2 = jnp.sum(cum2 >= n2[:, None], axis=1) - 1
    take2 = lambda a: jnp.take_along_axis(a, b2[:, None], axis=1)[:, 0]
    c2_above = take2(cum2) - take2(cnt2)
    s2_above = take2(_rev_cumsum(sum2)) - take2(sum2)
    ties = (n2 - c2_above).astype(jnp.float32)
    tie_mean = take2(sum2) / jnp.maximum(take2(cnt2), 1).astype(jnp.float32)

    row_total = s1_above + s2_above + ties * tie_mean
    return (jnp.sum(row_total) / jnp.float32(_BC * _N_TOP)).astype(jnp.float32)
